# baseline (speedup 1.0000x reference)
import functools

import jax
import jax.numpy as jnp
from jax import lax
from jax.experimental import pallas as pl
from jax.experimental.pallas import tpu as pltpu
from jax.experimental.pallas import tpu_sc as plsc

LANES = 16
CHUNKB = 8
UNROLL = 2
EPS = 1e-5


def _rsqrt_vec(x):
    i = plsc.bitcast(x, jnp.int32)
    i = jnp.int32(0x5F3759DF) - lax.shift_right_logical(i, 1)
    y = plsc.bitcast(i, jnp.float32)
    hx = 0.5 * x
    for _ in range(3):
        y = y * (1.5 - hx * y * y)
    return y


def _make_sc_kernel(batch, seq, hidden, max_pos):
    n_workers = 32
    bpw = batch // n_workers
    n_chunks = bpw // CHUNKB
    nk = hidden // LANES
    inv_h = 1.0 / hidden

    mesh = plsc.VectorSubcoreMesh(core_axis_name="c", subcore_axis_name="s")

    @functools.partial(
        pl.kernel,
        mesh=mesh,
        out_type=jax.ShapeDtypeStruct((batch, seq, hidden), jnp.float32),
        compiler_params=pltpu.CompilerParams(
            needs_layout_passes=False, use_tc_tiling_on_sc=False),
        scratch_types=[
            pltpu.VMEM((CHUNKB, seq), jnp.int32),
            pltpu.VMEM((CHUNKB, seq), jnp.int32),
            pltpu.VMEM((CHUNKB, seq, hidden), jnp.float32),
            pltpu.VMEM((CHUNKB, seq, hidden), jnp.float32),
            pltpu.VMEM((max_pos, hidden), jnp.float32),
            pltpu.VMEM((hidden,), jnp.float32),
            pltpu.VMEM((hidden,), jnp.float32),
            pltpu.SemaphoreType.DMA,
            pltpu.SemaphoreType.DMA,
        ],
    )
    def sc_kernel(ids_hbm, table_hbm, pos_hbm, gam_hbm, bet_hbm, out_hbm,
                  idx_v0, idx_v1, rows_v0, rows_v1, pos_v, gam_v, bet_v,
                  sem0, sem1):
        wid = lax.axis_index("s") * 2 + lax.axis_index("c")
        base = wid * bpw
        pltpu.sync_copy(pos_hbm, pos_v)
        pltpu.sync_copy(gam_hbm, gam_v)
        pltpu.sync_copy(bet_hbm, bet_v)
        idx_bufs = (idx_v0, idx_v1)
        row_bufs = (rows_v0, rows_v1)
        sems = (sem0, sem1)
        gs = [gam_v[pl.ds(k * LANES, LANES)] for k in range(nk)]
        bs = [bet_v[pl.ds(k * LANES, LANES)] for k in range(nk)]
        iota16 = lax.iota(jnp.int32, LANES)
        perms = [iota16 ^ d for d in (1, 2, 4, 8)]

        def _lane_sum(v):
            for pm in perms:
                v = v + v.at[pm].get(mode="promise_in_bounds")
            return v

        def start_gather(c, slot):
            b0 = base + c * CHUNKB
            pltpu.sync_copy(ids_hbm.at[pl.ds(b0, CHUNKB)], idx_bufs[slot])
            for b in range(CHUNKB):
                pltpu.async_copy(table_hbm.at[idx_bufs[slot].at[b]],
                                 row_bufs[slot].at[b], sems[slot])

        def process(c, slot):
            b0 = base + c * CHUNKB
            rows_v = row_bufs[slot]
            for b in range(CHUNKB):
                pltpu.make_async_copy(table_hbm.at[idx_bufs[slot].at[b]],
                                      rows_v.at[b], sems[slot]).wait()

            @plsc.parallel_loop(0, seq, unroll=UNROLL)
            def pos_body(p):
                pes = [pos_v[p, pl.ds(k * LANES, LANES)] for k in range(nk)]
                for b in range(CHUNKB):
                    xs = []
                    for k in range(nk):
                        xs.append(rows_v[b, p, pl.ds(k * LANES, LANES)]
                                  + pes[k])
                    s = (xs[0] + xs[1]) + (xs[2] + xs[3])
                    sq = (xs[0] * xs[0] + xs[1] * xs[1]) + \
                         (xs[2] * xs[2] + xs[3] * xs[3])
                    mean = _lane_sum(s) * inv_h
                    var = _lane_sum(sq) * inv_h - mean * mean
                    rstd = _rsqrt_vec(var + EPS)
                    nm = mean * rstd
                    for k in range(nk):
                        y = (xs[k] * rstd - nm) * gs[k] + bs[k]
                        rows_v[b, p, pl.ds(k * LANES, LANES)] = y

            pltpu.sync_copy(rows_v, out_hbm.at[pl.ds(b0, CHUNKB)])

        start_gather(jnp.int32(0), 0)

        def pair_body(i, carry):
            c0 = 2 * i
            start_gather(c0 + 1, 1)
            process(c0, 0)

            @pl.when(c0 + 2 < n_chunks)
            def _():
                start_gather(c0 + 2, 0)

            process(c0 + 1, 1)
            return carry

        lax.fori_loop(0, n_chunks // 2, pair_body, 0)

    return sc_kernel


def kernel(input_ids, token_table, pos_table, ln_gamma, ln_beta):
    b, s = input_ids.shape
    vocab, hidden = token_table.shape
    max_pos = pos_table.shape[0]
    ids = input_ids.astype(jnp.int32)
    sc = _make_sc_kernel(b, s, hidden, max_pos)
    return sc(ids, token_table, pos_table, ln_gamma, ln_beta)
